# baseline (speedup 1.0000x reference)
import functools

import jax
import jax.numpy as jnp
from jax import lax
from jax.experimental import pallas as pl
from jax.experimental.pallas import tpu as pltpu
from jax.experimental.pallas import tpu_sc as plsc

_K = 8192
_D = 64
_N = 16 * 576
_BETA = 0.25

_TB = 256
_NB = _N // _TB

_NC = 2
_NS = 16
_NW = _NC * _NS
_BPW = _N // _NW
_CH = 3
_CW = _BPW // _CH


def _dist_argmin_body(z_ref, emb_ref, codes_ref, loss_ref):
    i = pl.program_id(0)
    z = z_ref[...]
    emb = emb_ref[...]
    z_sq = jnp.sum(z * z, axis=1, keepdims=True)
    e_sq = jnp.sum(emb * emb, axis=1)[None, :]
    dot = lax.dot_general(z.astype(jnp.bfloat16), emb.astype(jnp.bfloat16),
                          (((1,), (1,)), ((), ())),
                          preferred_element_type=jnp.float32)
    dist = z_sq - 2.0 * dot + e_sq
    q = _K // 4
    run_v = run_i = run_true = None
    for c in range(4):
        seg = dist[:, c * q:(c + 1) * q]
        m = jnp.min(seg, axis=1)
        a = jnp.argmin(seg, axis=1).astype(jnp.int32) + jnp.int32(c * q)
        if c == 0:
            run_v = m.astype(jnp.bfloat16).astype(jnp.float32)
            run_i, run_true = a, m
        else:
            take = m < run_v
            run_i = jnp.where(take, a, run_i)
            run_true = jnp.where(take, m, run_true)
            run_v = jnp.where(take, m.astype(jnp.bfloat16).astype(jnp.float32),
                              run_v)
    codes_ref[...] = run_i
    partial = jnp.sum(run_true)

    @pl.when(i == 0)
    def _init():
        loss_ref[0] = 0.0

    loss_ref[0] += partial

    @pl.when(i == _NB - 1)
    def _fin():
        loss_ref[0] = loss_ref[0] * (_BETA / (_N * _D))


def _tc_codes_and_loss(z2d, emb):
    codes, loss = pl.pallas_call(
        _dist_argmin_body,
        grid=(_NB,),
        in_specs=[
            pl.BlockSpec((_TB, _D), lambda i: (i, 0)),
            pl.BlockSpec((_K, _D), lambda i: (0, 0)),
        ],
        out_specs=[
            pl.BlockSpec((_TB,), lambda i: (i,)),
            pl.BlockSpec(memory_space=pltpu.SMEM, block_shape=(1,),
                         index_map=lambda i: (0,)),
        ],
        out_shape=[
            jax.ShapeDtypeStruct((_N,), jnp.int32),
            jax.ShapeDtypeStruct((1,), jnp.float32),
        ],
    )(z2d, emb)
    return codes, loss[0]


def _sc_gather(emb, idx3d):
    mesh = plsc.VectorSubcoreMesh(core_axis_name="c", subcore_axis_name="s")

    @functools.partial(
        pl.kernel,
        mesh=mesh,
        out_type=jax.ShapeDtypeStruct((_N, _D), jnp.float32),
        scratch_types=[
            pltpu.VMEM((_CH, _CW), jnp.int32),
            pltpu.VMEM((_BPW, _D), jnp.float32),
            pltpu.SemaphoreType.DMA,
        ],
        compiler_params=pltpu.CompilerParams(use_tc_tiling_on_sc=False),
    )
    def gather_kernel(table_hbm, idx_hbm, out_hbm, idx_v, rows_v, sem):
        wid = lax.axis_index("s") * _NC + lax.axis_index("c")
        base = wid * _BPW
        pltpu.sync_copy(idx_hbm.at[wid], idx_v)
        copies = [
            pltpu.async_copy(table_hbm.at[idx_v.at[j]],
                             rows_v.at[pl.ds(j * _CW, _CW)], sem)
            for j in range(_CH)
        ]
        for c in copies:
            c.wait()
        pltpu.sync_copy(rows_v, out_hbm.at[pl.ds(base, _BPW)])

    return gather_kernel(emb, idx3d)


def kernel(z_e, embedding):
    orig_shape = z_e.shape
    z2d = z_e.reshape(-1, orig_shape[-1])
    codes, vq_loss = _tc_codes_and_loss(z2d, embedding)
    z_q = _sc_gather(embedding, codes.reshape(_NW, _CH, _CW))
    z_q_st = z_q.reshape(orig_shape)
    codes_out = codes.reshape(orig_shape[:-1])
    return (z_q_st, vq_loss, codes_out)


# speedup vs baseline: 1.4767x; 1.4767x over previous
import functools

import jax
import jax.numpy as jnp
from jax import lax
from jax.experimental import pallas as pl
from jax.experimental.pallas import tpu as pltpu
from jax.experimental.pallas import tpu_sc as plsc

_K = 8192
_D = 64
_N = 16 * 576
_BETA = 0.25

_TB = 512
_NB = _N // _TB

_NC = 2
_NS = 16
_NW = _NC * _NS
_BPW = _N // _NW
_CH = 3
_CW = _BPW // _CH


def _dist_argmin_body(z_ref, emb_ref, codes_ref, loss_ref):
    i = pl.program_id(0)
    z = z_ref[...]
    emb = emb_ref[...]
    z_sq = jnp.sum(z * z, axis=1, keepdims=True)
    e_sq = jnp.sum(emb * emb, axis=1)[None, :]
    dot = lax.dot_general(z.astype(jnp.bfloat16), emb.astype(jnp.bfloat16),
                          (((1,), (1,)), ((), ())),
                          preferred_element_type=jnp.float32)
    dist = z_sq - 2.0 * dot + e_sq
    q = _K // 4
    run_v = run_i = run_true = None
    for c in range(4):
        seg = dist[:, c * q:(c + 1) * q]
        m = jnp.min(seg, axis=1)
        a = jnp.argmin(seg, axis=1).astype(jnp.int32) + jnp.int32(c * q)
        if c == 0:
            run_v = m.astype(jnp.bfloat16).astype(jnp.float32)
            run_i, run_true = a, m
        else:
            take = m < run_v
            run_i = jnp.where(take, a, run_i)
            run_true = jnp.where(take, m, run_true)
            run_v = jnp.where(take, m.astype(jnp.bfloat16).astype(jnp.float32),
                              run_v)
    codes_ref[...] = run_i
    partial = jnp.sum(run_true)

    @pl.when(i == 0)
    def _init():
        loss_ref[0] = 0.0

    loss_ref[0] += partial

    @pl.when(i == _NB - 1)
    def _fin():
        loss_ref[0] = loss_ref[0] * (_BETA / (_N * _D))


def _tc_codes_and_loss(z2d, emb):
    codes, loss = pl.pallas_call(
        _dist_argmin_body,
        grid=(_NB,),
        in_specs=[
            pl.BlockSpec((_TB, _D), lambda i: (i, 0)),
            pl.BlockSpec((_K, _D), lambda i: (0, 0)),
        ],
        out_specs=[
            pl.BlockSpec((_TB,), lambda i: (i,)),
            pl.BlockSpec(memory_space=pltpu.SMEM, block_shape=(1,),
                         index_map=lambda i: (0,)),
        ],
        out_shape=[
            jax.ShapeDtypeStruct((_N,), jnp.int32),
            jax.ShapeDtypeStruct((1,), jnp.float32),
        ],
    )(z2d, emb)
    return codes, loss[0]


def _sc_gather(emb, idx3d):
    mesh = plsc.VectorSubcoreMesh(core_axis_name="c", subcore_axis_name="s")

    @functools.partial(
        pl.kernel,
        mesh=mesh,
        out_type=jax.ShapeDtypeStruct((_N, _D), jnp.float32),
        scratch_types=[
            pltpu.VMEM((_CH, _CW), jnp.int32),
            pltpu.VMEM((_BPW, _D), jnp.float32),
            pltpu.SemaphoreType.DMA,
        ],
        compiler_params=pltpu.CompilerParams(use_tc_tiling_on_sc=False),
    )
    def gather_kernel(table_hbm, idx_hbm, out_hbm, idx_v, rows_v, sem):
        wid = lax.axis_index("s") * _NC + lax.axis_index("c")
        base = wid * _BPW
        pltpu.sync_copy(idx_hbm.at[wid], idx_v)
        copies = [
            pltpu.async_copy(table_hbm.at[idx_v.at[j]],
                             rows_v.at[pl.ds(j * _CW, _CW)], sem)
            for j in range(_CH)
        ]
        for c in copies:
            c.wait()
        pltpu.sync_copy(rows_v, out_hbm.at[pl.ds(base, _BPW)])

    return gather_kernel(emb, idx3d)


def kernel(z_e, embedding):
    orig_shape = z_e.shape
    z2d = z_e.reshape(-1, orig_shape[-1])
    codes, vq_loss = _tc_codes_and_loss(z2d, embedding)
    z_q = _sc_gather(embedding, codes.reshape(_NW, _CH, _CW))
    z_q_st = z_q.reshape(orig_shape)
    codes_out = codes.reshape(orig_shape[:-1])
    return (z_q_st, vq_loss, codes_out)
